# baseline (speedup 1.0000x reference)
import functools

import jax
import jax.numpy as jnp
from jax import lax
from jax.experimental import pallas as pl
from jax.experimental.pallas import tpu as pltpu
from jax.experimental.pallas import tpu_sc as plsc

_NG = 2
_H = 64
_W = 64
_HW = _H * _W
_NSPOT = 50
_LANES = 16
_LN2 = 0.6931471805599453
_NSC = 10
_TPS = 3
_PSPOT = 17
_GROUPS = (10, 7)


def _vlog(x):
    xi = lax.bitcast_convert_type(x, jnp.int32)
    e = lax.shift_right_arithmetic(xi, 23) - 127
    m = lax.bitcast_convert_type((xi & 0x007FFFFF) | 0x3F800000, jnp.float32)
    big = m > 1.4142135623730951
    m = jnp.where(big, m * 0.5, m)
    e = jnp.where(big, e + 1, e).astype(jnp.float32)
    t = (m - 1.0) / (m + 1.0)
    t2 = t * t
    p = 2.0 + t2 * (2.0 / 3.0 + t2 * (2.0 / 5.0 + t2 * (2.0 / 7.0 + t2 * (2.0 / 9.0))))
    return e * _LN2 + t * p


_GATHER_DNUMS = lax.GatherDimensionNumbers(
    offset_dims=(), collapsed_slice_dims=(0,), start_index_map=(0,))


def _shuffle(x, idx):
    return lax.gather(x, idx[:, None], _GATHER_DNUMS, (1,),
                      mode=lax.GatherScatterMode.PROMISE_IN_BOUNDS)


def _hreduce(x, op):
    idx = lax.iota(jnp.int32, _LANES)
    for k in (1, 2, 4, 8):
        x = op(x, _shuffle(x, idx ^ k))
    return x


def _splat_spot(ref, row, s):
    chunk = lax.shift_right_logical(s, 4)
    lane = lax.bitwise_and(s, 15)
    vec = ref[row, pl.ds(lax.shift_left(chunk, 4), _LANES)]
    return _shuffle(vec, jnp.full((_LANES,), lane, jnp.int32))


def _gmm_sc_body(feat_hbm, tgt_hbm, out_hbm, feat_v, tgt_v, r1_v, r2_v, c_v, out_v):
    cid = lax.axis_index("c")
    sid = lax.axis_index("s")
    wid = sid * 2 + cid
    slice_id = wid // _TPS
    part = wid - slice_id * _TPS
    valid_tile = slice_id <= _NSC
    slice_eff = jnp.minimum(slice_id, _NSC)
    b = slice_eff // 8
    f = slice_eff - b * 8
    sbase = part * _PSPOT
    s_end = jnp.where(valid_tile, jnp.minimum(sbase + _PSPOT, _NSPOT), 0)

    pltpu.sync_copy(feat_hbm.at[b, f], feat_v)
    pltpu.sync_copy(tgt_hbm.at[slice_eff], tgt_v)

    zero = jnp.zeros((_LANES,), jnp.float32)

    def _rc(i):
        r = lax.shift_right_logical(i, 2)
        col = pl.ds(lax.shift_left(lax.bitwise_and(i, 3), 4), _LANES)
        return r, col

    def prep(ii, sp):
        for u in range(2):
            i = ii * 2 + u
            r, col = _rc(i)
            sl = pl.ds(i * _LANES, _LANES)
            p = jnp.maximum(feat_v[0, r, col], 1e-20)
            s1 = jnp.maximum(feat_v[3, r, col], 1e-10)
            s2 = jnp.maximum(feat_v[4, r, col], 1e-10)
            inv1 = 1.0 / s1
            inv2 = 1.0 / s2
            r1_v[sl] = (0.5 * inv1) * inv1
            r2_v[sl] = (0.5 * inv2) * inv2
            c_v[sl] = (p * inv1) * inv2
            sp = sp + p
        return sp

    nchunk = _HW // _LANES
    sp = lax.fori_loop(0, nchunk // 2, prep, zero)
    kshift_v = zero - _vlog(_hreduce(sp, jnp.add))

    loss_v = zero
    msum_v = zero
    g0 = 0
    for gsize in _GROUPS:
        spots = [sbase + s for s in range(g0, g0 + gsize)]
        g0 += gsize
        tv1 = [_splat_spot(tgt_v, 1, s) for s in spots]
        tv2 = [_splat_spot(tgt_v, 2, s) for s in spots]

        def body(i, accs):
            accs = list(accs)
            r, col = _rc(i)
            sl = pl.ds(i * _LANES, _LANES)
            c = c_v[sl]
            r1 = r1_v[sl]
            r2 = r2_v[sl]
            m1 = feat_v[1, r, col]
            m2 = feat_v[2, r, col]
            for j in range(len(spots)):
                d1 = tv1[j] - m1
                d2 = tv2[j] - m2
                accs[j] = accs[j] + c * jnp.exp(-(r1 * (d1 * d1) + r2 * (d2 * d2)))
            return tuple(accs)

        accs = lax.fori_loop(0, nchunk, body, tuple(zero for _ in spots))
        for j, s in enumerate(spots):
            ssum = jnp.maximum(_hreduce(accs[j], jnp.add), 1e-37)
            logv = _vlog(ssum)
            validf = jnp.where(s < s_end, 1.0, 0.0)
            mask = _splat_spot(tgt_v, 0, s) * jnp.full((_LANES,), validf)
            loss_v = loss_v + mask * logv
            msum_v = msum_v + mask

    out_v[pl.ds(0, _LANES)] = -(loss_v + msum_v * kshift_v)
    pltpu.sync_copy(out_v, out_hbm.at[wid])


_LOG2E = 1.4426950408889634


def _gmm_tc_body(feat_ref, tgt_ref, out_ref):
    feat = feat_ref[0]
    mask = tgt_ref[0, 0, :_NSPOT]
    tv1 = tgt_ref[0, 1, :_NSPOT]
    tv2 = tgt_ref[0, 2, :_NSPOT]
    p = jnp.maximum(feat[0], 1e-20)
    mu1 = feat[1]
    mu2 = feat[2]
    s1 = jnp.maximum(feat[3], 1e-10)
    s2 = jnp.maximum(feat[4], 1e-10)
    r1 = (0.5 * _LOG2E) / (s1 * s1)
    r2 = (0.5 * _LOG2E) / (s2 * s2)
    c = jnp.log2(p / (s1 * s2))
    mhat = jnp.max(c)
    logz = jnp.log2(jnp.sum(p))
    cs = c - mhat
    d1 = tv1[:, None, None] - mu1[None]
    d2 = tv2[:, None, None] - mu2[None]
    e = jnp.exp2(cs[None] - r1[None] * (d1 * d1) - r2[None] * (d2 * d2))
    ssum = jnp.maximum(jnp.sum(e, axis=(1, 2)), 1e-37)
    loss = -_LN2 * jnp.sum(mask * (jnp.log2(ssum) + mhat - logz))
    out_ref[0, 0, :] = jnp.full((128,), loss, jnp.float32)


@jax.jit
def _gmm_call(feats, feats32, tgt3, tgt_t):
    sc = pl.kernel(
        _gmm_sc_body,
        out_type=jax.ShapeDtypeStruct((32, _LANES), jnp.float32),
        mesh=plsc.VectorSubcoreMesh(core_axis_name="c", subcore_axis_name="s"),
        scratch_types=[
            pltpu.VMEM((5, _H, _W), jnp.float32),
            pltpu.VMEM((3, 128), jnp.float32),
            pltpu.VMEM((_HW,), jnp.float32),
            pltpu.VMEM((_HW,), jnp.float32),
            pltpu.VMEM((_HW,), jnp.float32),
            pltpu.VMEM((_LANES,), jnp.float32),
        ],
    )
    out_sc = sc(feats, tgt_t)

    ntc = 32 - _NSC
    out_tc = pl.pallas_call(
        _gmm_tc_body,
        grid=(ntc,),
        in_specs=[
            pl.BlockSpec((1, 5, _H, _W), lambda i: (i + _NSC, 0, 0, 0)),
            pl.BlockSpec((1, 3, 128), lambda i: (i + _NSC, 0, 0)),
        ],
        out_specs=pl.BlockSpec((1, 1, 128), lambda i: (i, 0, 0)),
        out_shape=jax.ShapeDtypeStruct((ntc, 1, 128), jnp.float32),
    )(feats32, tgt_t.at[_NSC, 0, :2 * _PSPOT].set(0.0))

    loss_sc = out_sc[:_NSC * _TPS, 0].reshape(_NSC, _TPS).sum(axis=1)
    loss_tc = out_tc[:, 0, 0]
    loss_tc = loss_tc.at[0].add(out_sc[_NSC * _TPS, 0] + out_sc[_NSC * _TPS + 1, 0])
    return jnp.concatenate([loss_sc, loss_tc])


def kernel(features, targets):
    B, F, nf, h, w = features.shape
    max_spots = targets.shape[2]
    assert nf == 2 * _NG + 1 and h == _H and w == _W
    assert B * F == 32 and max_spots == _NSPOT
    feats32 = features.reshape(B * F, nf, h, w)
    tgt3 = targets.reshape(B * F, max_spots, _NG + 1)
    tgt_t = jnp.pad(tgt3.transpose(0, 2, 1), ((0, 0), (0, 0), (0, 128 - max_spots)))
    out = _gmm_call(features, feats32, tgt3, tgt_t)
    return out.reshape(B, F)


# speedup vs baseline: 1.6027x; 1.6027x over previous
import functools

import jax
import jax.numpy as jnp
from jax import lax
from jax.experimental import pallas as pl
from jax.experimental.pallas import tpu as pltpu
from jax.experimental.pallas import tpu_sc as plsc

_NG = 2
_H = 64
_W = 64
_HW = _H * _W
_NSPOT = 50
_LANES = 16
_LN2 = 0.6931471805599453
_NSC = 10
_TPS = 3
_PSPOT = 17
_GROUPS = (10, 7)


def _vlog(x):
    xi = lax.bitcast_convert_type(x, jnp.int32)
    e = lax.shift_right_arithmetic(xi, 23) - 127
    m = lax.bitcast_convert_type((xi & 0x007FFFFF) | 0x3F800000, jnp.float32)
    big = m > 1.4142135623730951
    m = jnp.where(big, m * 0.5, m)
    e = jnp.where(big, e + 1, e).astype(jnp.float32)
    t = (m - 1.0) / (m + 1.0)
    t2 = t * t
    p = 2.0 + t2 * (2.0 / 3.0 + t2 * (2.0 / 5.0 + t2 * (2.0 / 7.0 + t2 * (2.0 / 9.0))))
    return e * _LN2 + t * p


_GATHER_DNUMS = lax.GatherDimensionNumbers(
    offset_dims=(), collapsed_slice_dims=(0,), start_index_map=(0,))


def _shuffle(x, idx):
    return lax.gather(x, idx[:, None], _GATHER_DNUMS, (1,),
                      mode=lax.GatherScatterMode.PROMISE_IN_BOUNDS)


def _hreduce(x, op):
    idx = lax.iota(jnp.int32, _LANES)
    for k in (1, 2, 4, 8):
        x = op(x, _shuffle(x, idx ^ k))
    return x


def _splat_spot(ref, row, s):
    chunk = lax.shift_right_logical(s, 4)
    lane = lax.bitwise_and(s, 15)
    vec = ref[row, pl.ds(lax.shift_left(chunk, 4), _LANES)]
    return _shuffle(vec, jnp.full((_LANES,), lane, jnp.int32))


def _gmm_sc_body(feat_hbm, tgt_hbm, out_hbm, feat_v, tgt_v, r1_v, r2_v, c_v, out_v):
    cid = lax.axis_index("c")
    sid = lax.axis_index("s")
    wid = sid * 2 + cid
    slice_id = wid // _TPS
    part = wid - slice_id * _TPS
    valid_tile = slice_id < _NSC
    slice_eff = jnp.minimum(slice_id, _NSC - 1)
    b = slice_eff // 8
    f = slice_eff - b * 8
    sbase = part * _PSPOT
    s_end = jnp.where(valid_tile, jnp.minimum(sbase + _PSPOT, _NSPOT), 0)

    pltpu.sync_copy(feat_hbm.at[b, f], feat_v)
    pltpu.sync_copy(tgt_hbm.at[slice_eff], tgt_v)

    zero = jnp.zeros((_LANES,), jnp.float32)

    def _rc(i):
        r = lax.shift_right_logical(i, 2)
        col = pl.ds(lax.shift_left(lax.bitwise_and(i, 3), 4), _LANES)
        return r, col

    def prep(ii, sp):
        for u in range(2):
            i = ii * 2 + u
            r, col = _rc(i)
            sl = pl.ds(i * _LANES, _LANES)
            p = jnp.maximum(feat_v[0, r, col], 1e-20)
            s1 = jnp.maximum(feat_v[3, r, col], 1e-10)
            s2 = jnp.maximum(feat_v[4, r, col], 1e-10)
            inv1 = 1.0 / s1
            inv2 = 1.0 / s2
            r1_v[sl] = (0.5 * inv1) * inv1
            r2_v[sl] = (0.5 * inv2) * inv2
            c_v[sl] = (p * inv1) * inv2
            sp = sp + p
        return sp

    nchunk = _HW // _LANES
    sp = lax.fori_loop(0, nchunk // 2, prep, zero)
    kshift_v = zero - _vlog(_hreduce(sp, jnp.add))

    loss_v = zero
    msum_v = zero
    g0 = 0
    for gsize in _GROUPS:
        spots = [sbase + s for s in range(g0, g0 + gsize)]
        g0 += gsize
        tv1 = [_splat_spot(tgt_v, 1, s) for s in spots]
        tv2 = [_splat_spot(tgt_v, 2, s) for s in spots]

        def body(i, accs):
            accs = list(accs)
            r, col = _rc(i)
            sl = pl.ds(i * _LANES, _LANES)
            c = c_v[sl]
            r1 = r1_v[sl]
            r2 = r2_v[sl]
            m1 = feat_v[1, r, col]
            m2 = feat_v[2, r, col]
            for j in range(len(spots)):
                d1 = tv1[j] - m1
                d2 = tv2[j] - m2
                accs[j] = accs[j] + c * jnp.exp(-(r1 * (d1 * d1) + r2 * (d2 * d2)))
            return tuple(accs)

        accs = lax.fori_loop(0, nchunk, body, tuple(zero for _ in spots))
        for j, s in enumerate(spots):
            ssum = jnp.maximum(_hreduce(accs[j], jnp.add), 1e-37)
            logv = _vlog(ssum)
            validf = jnp.where(s < s_end, 1.0, 0.0)
            mask = _splat_spot(tgt_v, 0, s) * jnp.full((_LANES,), validf)
            loss_v = loss_v + mask * logv
            msum_v = msum_v + mask

    out_v[pl.ds(0, _LANES)] = -(loss_v + msum_v * kshift_v)
    pltpu.sync_copy(out_v, out_hbm.at[wid])


_LOG2E = 1.4426950408889634


def _gmm_tc_body(feat_ref, tgt_ref, out_ref):
    feat = feat_ref[0]
    mask = tgt_ref[0, 0, :_NSPOT]
    tv1 = tgt_ref[0, 1, :_NSPOT]
    tv2 = tgt_ref[0, 2, :_NSPOT]
    p = jnp.maximum(feat[0], 1e-20)
    mu1 = feat[1]
    mu2 = feat[2]
    s1 = jnp.maximum(feat[3], 1e-10)
    s2 = jnp.maximum(feat[4], 1e-10)
    r1 = (0.5 * _LOG2E) / (s1 * s1)
    r2 = (0.5 * _LOG2E) / (s2 * s2)
    c = jnp.log2(p / (s1 * s2))
    mhat = jnp.max(c)
    logz = jnp.log2(jnp.sum(p))
    cs = c - mhat
    d1 = tv1[:, None, None] - mu1[None]
    d2 = tv2[:, None, None] - mu2[None]
    e = jnp.exp2(cs[None] - r1[None] * (d1 * d1) - r2[None] * (d2 * d2))
    ssum = jnp.maximum(jnp.sum(e, axis=(1, 2)), 1e-37)
    loss = -_LN2 * jnp.sum(mask * (jnp.log2(ssum) + mhat - logz))
    out_ref[0, 0, :] = jnp.full((128,), loss, jnp.float32)


@jax.jit
def _gmm_call(feats, feats32, tgt3, tgt_t):
    sc = pl.kernel(
        _gmm_sc_body,
        out_type=jax.ShapeDtypeStruct((32, _LANES), jnp.float32),
        mesh=plsc.VectorSubcoreMesh(core_axis_name="c", subcore_axis_name="s"),
        scratch_types=[
            pltpu.VMEM((5, _H, _W), jnp.float32),
            pltpu.VMEM((3, 128), jnp.float32),
            pltpu.VMEM((_HW,), jnp.float32),
            pltpu.VMEM((_HW,), jnp.float32),
            pltpu.VMEM((_HW,), jnp.float32),
            pltpu.VMEM((_LANES,), jnp.float32),
        ],
    )
    out_sc = sc(feats, tgt_t)

    ntc = 32 - _NSC
    out_tc = pl.pallas_call(
        _gmm_tc_body,
        grid=(ntc,),
        in_specs=[
            pl.BlockSpec((1, 5, _H, _W), lambda i: (i + _NSC, 0, 0, 0)),
            pl.BlockSpec((1, 3, 128), lambda i: (i + _NSC, 0, 0)),
        ],
        out_specs=pl.BlockSpec((1, 1, 128), lambda i: (i, 0, 0)),
        out_shape=jax.ShapeDtypeStruct((ntc, 1, 128), jnp.float32),
    )(feats32, tgt_t)

    loss_sc = out_sc[:_NSC * _TPS, 0].reshape(_NSC, _TPS).sum(axis=1)
    loss_tc = out_tc[:, 0, 0]
    return jnp.concatenate([loss_sc, loss_tc])


def kernel(features, targets):
    B, F, nf, h, w = features.shape
    max_spots = targets.shape[2]
    assert nf == 2 * _NG + 1 and h == _H and w == _W
    assert B * F == 32 and max_spots == _NSPOT
    feats32 = features.reshape(B * F, nf, h, w)
    tgt3 = targets.reshape(B * F, max_spots, _NG + 1)
    tgt_t = jnp.pad(tgt3.transpose(0, 2, 1), ((0, 0), (0, 0), (0, 128 - max_spots)))
    out = _gmm_call(features, feats32, tgt3, tgt_t)
    return out.reshape(B, F)
